# baseline (speedup 1.0000x reference)
import functools

import jax
import jax.numpy as jnp
from jax import lax
from jax.experimental import pallas as pl
from jax.experimental.pallas import tpu as pltpu
from jax.experimental.pallas import tpu_sc as plsc

N_WORDS = 2048
N_SAMPLES = 4096
PATH_LEN = 256

NC = 2
NS = 16
NW = NC * NS

EPW = (N_SAMPLES * PATH_LEN) // NW
ROWS = EPW // 128
VPR = 8
HALF = 8
SUPER = ROWS // 16
SE_PW = N_SAMPLES // NW
SROWS = SE_PW
SROWS = SE_PW
DIAG_PW = N_WORDS // NW

_INV = 1.0 / N_SAMPLES


def _add_body(x_ref, y_ref, o_ref):
    o_ref[...] = (x_ref[...] + y_ref[...]).reshape(128, 16, 128)


def _tc_add(x, y):
    return pl.pallas_call(
        _add_body,
        grid=(16,),
        in_specs=[pl.BlockSpec((128, N_WORDS), lambda i: (i, 0))] * 2,
        out_specs=pl.BlockSpec((128, 16, 128), lambda i: (i, 0, 0)),
        out_shape=jax.ShapeDtypeStruct((N_WORDS, 16, 128), jnp.float32),
    )(x, y)


def _sc_body(bg_hbm, start_hbm, end_hbm, s_hbm,
             out_hbm,
             sv2, idx_v, gb_v, s0i_v, e0i_v, sg_v, eg_v,
             di_v, dg_v, w0_v, w1_v, o_v,
             semA, semB, semT):
    wid = lax.axis_index("s") * NC + lax.axis_index("c")
    iota = lax.iota(jnp.int32, 16)
    zeros16 = jnp.zeros((16,), jnp.int32)

    pltpu.sync_copy(s_hbm.at[pl.ds(wid * SROWS, SROWS)], sv2)

    dims1d = lax.GatherDimensionNumbers(offset_dims=(), collapsed_slice_dims=(0,),
                                        start_index_map=(0,))

    def lane_pick(v, idx_vec):
        return lax.gather(v, idx_vec[:, None], dimension_numbers=dims1d,
                          slice_sizes=(1,),
                          mode=lax.GatherScatterMode.PROMISE_IN_BOUNDS)

    rot = (iota + 1) & 15
    for j in range(SROWS // 16):
        s0i_v[pl.ds(j * 16, 16)] = zeros16
        e0i_v[pl.ds(j * 16, 16)] = zeros16

    def idx_body(r, carry):
        lane = r & 15
        seg = pl.ds((r >> 4) * 16, 16)
        for c in range(16):
            va = sv2[r, pl.ds(c * 16, 16)]
            if c == 0:
                contrib = jnp.where(iota == lane, lane_pick(va, zeros16), 0)
                s0i_v[seg] = s0i_v[seg] + contrib
            if c < 15:
                vb = sv2[r, pl.ds(c * 16 + 1, 16)]
            else:
                contrib = jnp.where(iota == lane, lane_pick(va, zeros16 + 15), 0)
                e0i_v[seg] = e0i_v[seg] + contrib
                vb = lane_pick(va, rot)
            idx_v[2 * r + (c // 8), pl.ds((c % 8) * 16, 16)] = \
                (va & (N_WORDS - 1)) * N_WORDS + (vb & (N_WORDS - 1))
        return carry
    lax.fori_loop(0, SROWS, idx_body, 0)

    def fire(row, slot, sem):
        pltpu.async_copy(bg_hbm.at[idx_v.at[row]], gb_v.at[slot], sem)

    def drain_half(sem, lo):
        for k in range(HALF):
            pltpu.make_async_copy(bg_hbm.at[pl.ds(0, 128)], gb_v.at[lo + k], sem).wait()

    for j in range(DIAG_PW // 16):
        i_vec = wid * DIAG_PW + j * 16 + iota
        valid = i_vec < N_WORDS - 1
        di_v[pl.ds(j * 16, 16)] = jnp.where(valid, i_vec * (N_WORDS + 1) + 1, 0)
    pltpu.async_copy(bg_hbm.at[di_v], dg_v, semT)
    pltpu.async_copy(start_hbm.at[s0i_v], sg_v, semT)
    pltpu.async_copy(end_hbm.at[e0i_v], eg_v, semT)
    pltpu.async_copy(start_hbm.at[zeros16], w0_v, semT)
    pltpu.async_copy(end_hbm.at[zeros16 + (N_WORDS - 1)], w1_v, semT)

    for k in range(HALF):
        fire(k, k, semA)
    for k in range(HALF):
        fire(HALF + k, HALF + k, semB)

    mask7 = jnp.where(iota < 15, 1.0, 0.0).astype(jnp.float32)

    def acc_slot(acc, k):
        for c in range(VPR - 1):
            acc = acc + gb_v[k, pl.ds(c * 16, 16)]
        last = gb_v[k, pl.ds(112, 16)]
        if k % 2 == 1:
            last = last * mask7
        return acc + last

    def gather_body(g, acc):
        drain_half(semA, 0)
        for k in range(HALF):
            acc = acc_slot(acc, k)

        @pl.when(g < SUPER - 1)
        def _():
            for k in range(HALF):
                fire((g + 1) * 16 + k, k, semA)

        drain_half(semB, HALF)
        for k in range(HALF):
            acc = acc_slot(acc, HALF + k)

        @pl.when(g < SUPER - 1)
        def _():
            for k in range(HALF):
                fire((g + 1) * 16 + 8 + k, HALF + k, semB)

        return acc

    acc = lax.fori_loop(0, SUPER, gather_body, jnp.zeros((16,), jnp.float32))

    pltpu.make_async_copy(bg_hbm.at[pl.ds(0, DIAG_PW)], dg_v, semT).wait()
    pltpu.make_async_copy(start_hbm.at[pl.ds(0, SE_PW)], sg_v, semT).wait()
    pltpu.make_async_copy(end_hbm.at[pl.ds(0, SE_PW)], eg_v, semT).wait()
    pltpu.make_async_copy(start_hbm.at[pl.ds(0, 16)], w0_v, semT).wait()
    pltpu.make_async_copy(end_hbm.at[pl.ds(0, 16)], w1_v, semT).wait()
    for c in range(SE_PW // 16):
        acc = acc + sg_v[pl.ds(c * 16, 16)] + eg_v[pl.ds(c * 16, 16)]

    accn = jnp.zeros((16,), jnp.float32)
    for j in range(DIAG_PW // 16):
        i_vec = wid * DIAG_PW + j * 16 + iota
        m = jnp.where(i_vec < N_WORDS - 1, 1.0, 0.0).astype(jnp.float32)
        accn = accn + dg_v[pl.ds(j * 16, 16)] * m

    lane0 = jnp.where(iota == 0, 1.0, 0.0).astype(jnp.float32)
    wmask = jnp.where(wid == 0, 1.0, 0.0).astype(jnp.float32)
    accn = accn + (w0_v[...] + w1_v[...]) * lane0 * wmask

    o_v[...] = acc * _INV - accn
    pltpu.sync_copy(o_v, out_hbm.at[pl.ds(wid * 16, 16)])


@jax.jit
def _sc_loss(bg, start, end, s):
    mesh = plsc.VectorSubcoreMesh(core_axis_name="c", subcore_axis_name="s",
                                  num_cores=NC, num_subcores=NS)
    grid_kernel = pl.kernel(
        _sc_body,
        compiler_params=pltpu.CompilerParams(use_tc_tiling_on_sc=True),
        out_type=jax.ShapeDtypeStruct((NW * 16,), jnp.float32),
        mesh=mesh,
        scratch_types=[
            pltpu.VMEM((SROWS, PATH_LEN), jnp.int32),
            pltpu.VMEM((ROWS, 128), jnp.int32),
            pltpu.VMEM((2 * HALF, 128), jnp.float32),
            pltpu.VMEM((SE_PW,), jnp.int32),
            pltpu.VMEM((SE_PW,), jnp.int32),
            pltpu.VMEM((SE_PW,), jnp.float32),
            pltpu.VMEM((SE_PW,), jnp.float32),
            pltpu.VMEM((DIAG_PW,), jnp.int32),
            pltpu.VMEM((DIAG_PW,), jnp.float32),
            pltpu.VMEM((16,), jnp.float32),
            pltpu.VMEM((16,), jnp.float32),
            pltpu.VMEM((16,), jnp.float32),
            pltpu.SemaphoreType.DMA,
            pltpu.SemaphoreType.DMA,
            pltpu.SemaphoreType.DMA,
        ],
    )
    return grid_kernel(bg, start, end, s)


def kernel(bigram, start, end, bigram_bias, samples):
    big = _tc_add(bigram, bigram_bias)
    partials = _sc_loss(big.reshape(-1), start, end,
                        samples.astype(jnp.int32))
    loss = jnp.sum(partials)
    return (loss, 0)


# speedup vs baseline: 1.0090x; 1.0090x over previous
import functools

import jax
import jax.numpy as jnp
from jax import lax
from jax.experimental import pallas as pl
from jax.experimental.pallas import tpu as pltpu
from jax.experimental.pallas import tpu_sc as plsc

N_WORDS = 2048
N_SAMPLES = 4096
PATH_LEN = 256

NC = 2
NS = 16
NW = NC * NS

EPW = (N_SAMPLES * PATH_LEN) // NW
ROWS = EPW // 128
VPR = 8
HALF = 4
SUPER = ROWS // (2 * HALF)
SE_PW = N_SAMPLES // NW
SROWS = SE_PW
SROWS = SE_PW
DIAG_PW = N_WORDS // NW

_INV = 1.0 / N_SAMPLES


def _add_body(x_ref, y_ref, o_ref):
    o_ref[...] = (x_ref[...] + y_ref[...]).reshape(128, 16, 128)


def _tc_add(x, y):
    return pl.pallas_call(
        _add_body,
        grid=(16,),
        in_specs=[pl.BlockSpec((128, N_WORDS), lambda i: (i, 0))] * 2,
        out_specs=pl.BlockSpec((128, 16, 128), lambda i: (i, 0, 0)),
        out_shape=jax.ShapeDtypeStruct((N_WORDS, 16, 128), jnp.float32),
    )(x, y)


def _sc_body(bg_hbm, start_hbm, end_hbm, s_hbm,
             out_hbm,
             sv, idx_v, gb_v, s0i_v, e0i_v, sg_v, eg_v, ps_v, pe_v,
             di_v, dg_v, w0_v, w1_v, o_v,
             semA, semB, semT):
    wid = lax.axis_index("s") * NC + lax.axis_index("c")
    iota = lax.iota(jnp.int32, 16)
    zeros16 = jnp.zeros((16,), jnp.int32)

    pltpu.sync_copy(s_hbm.at[pl.ds(wid * EPW, EPW)], sv.at[pl.ds(0, EPW)])
    sv[pl.ds(EPW, 16)] = zeros16

    def idx_body(r, carry):
        for c in range(VPR):
            base = r * 128 + c * 16
            va = sv[pl.ds(base, 16)]
            vb = sv[pl.ds(base + 1, 16)]
            idx_v[r, pl.ds(c * 16, 16)] = va * N_WORDS + vb
        return carry
    lax.fori_loop(0, ROWS, idx_body, 0)

    def fire(row, slot, sem):
        pltpu.async_copy(bg_hbm.at[idx_v.at[row]], gb_v.at[slot], sem)

    def drain_half(sem, lo):
        for k in range(HALF):
            pltpu.make_async_copy(bg_hbm.at[pl.ds(0, 128)], gb_v.at[lo + k], sem).wait()

    for j in range(SROWS // 16):
        pos = wid * EPW + (j * 16 + iota) * PATH_LEN
        ps_v[pl.ds(j * 16, 16)] = pos
        pe_v[pl.ds(j * 16, 16)] = pos + (PATH_LEN - 1)
    for j in range(DIAG_PW // 16):
        i_vec = wid * DIAG_PW + j * 16 + iota
        valid = i_vec < N_WORDS - 1
        di_v[pl.ds(j * 16, 16)] = jnp.where(valid, i_vec * (N_WORDS + 1) + 1, 0)
    pltpu.async_copy(s_hbm.at[ps_v], s0i_v, semT)
    pltpu.async_copy(s_hbm.at[pe_v], e0i_v, semT)
    pltpu.async_copy(bg_hbm.at[di_v], dg_v, semT)
    pltpu.async_copy(start_hbm.at[zeros16], w0_v, semT)
    pltpu.async_copy(end_hbm.at[zeros16 + (N_WORDS - 1)], w1_v, semT)

    for k in range(HALF):
        fire(k, k, semA)
    for k in range(HALF):
        fire(HALF + k, HALF + k, semB)

    mask7 = jnp.where(iota < 15, 1.0, 0.0).astype(jnp.float32)

    def acc_slot(acc, k):
        for c in range(VPR - 1):
            acc = acc + gb_v[k, pl.ds(c * 16, 16)]
        last = gb_v[k, pl.ds(112, 16)]
        if k % 2 == 1:
            last = last * mask7
        return acc + last

    def gather_body(g, acc):
        drain_half(semA, 0)
        for k in range(HALF):
            acc = acc_slot(acc, k)

        @pl.when(g < SUPER - 1)
        def _():
            for k in range(HALF):
                fire((g + 1) * 2 * HALF + k, k, semA)

        drain_half(semB, HALF)
        for k in range(HALF):
            acc = acc_slot(acc, HALF + k)

        @pl.when(g < SUPER - 1)
        def _():
            for k in range(HALF):
                fire((g + 1) * 2 * HALF + HALF + k, HALF + k, semB)

        return acc

    acc = lax.fori_loop(0, SUPER, gather_body, jnp.zeros((16,), jnp.float32))

    pltpu.make_async_copy(start_hbm.at[pl.ds(0, SE_PW)], s0i_v, semT).wait()
    pltpu.make_async_copy(start_hbm.at[pl.ds(0, SE_PW)], e0i_v, semT).wait()
    pltpu.make_async_copy(bg_hbm.at[pl.ds(0, DIAG_PW)], dg_v, semT).wait()
    pltpu.make_async_copy(start_hbm.at[pl.ds(0, 16)], w0_v, semT).wait()
    pltpu.make_async_copy(end_hbm.at[pl.ds(0, 16)], w1_v, semT).wait()
    h1 = pltpu.async_copy(start_hbm.at[s0i_v], sg_v, semA)
    h2 = pltpu.async_copy(end_hbm.at[e0i_v], eg_v, semB)
    h1.wait()
    h2.wait()
    for c in range(SE_PW // 16):
        acc = acc + sg_v[pl.ds(c * 16, 16)] + eg_v[pl.ds(c * 16, 16)]

    accn = jnp.zeros((16,), jnp.float32)
    for j in range(DIAG_PW // 16):
        i_vec = wid * DIAG_PW + j * 16 + iota
        m = jnp.where(i_vec < N_WORDS - 1, 1.0, 0.0).astype(jnp.float32)
        accn = accn + dg_v[pl.ds(j * 16, 16)] * m

    lane0 = jnp.where(iota == 0, 1.0, 0.0).astype(jnp.float32)
    wmask = jnp.where(wid == 0, 1.0, 0.0).astype(jnp.float32)
    accn = accn + (w0_v[...] + w1_v[...]) * lane0 * wmask

    o_v[...] = acc * _INV - accn
    pltpu.sync_copy(o_v, out_hbm.at[pl.ds(wid * 16, 16)])


@jax.jit
def _sc_loss(bg, start, end, s):
    mesh = plsc.VectorSubcoreMesh(core_axis_name="c", subcore_axis_name="s",
                                  num_cores=NC, num_subcores=NS)
    grid_kernel = pl.kernel(
        _sc_body,
        out_type=jax.ShapeDtypeStruct((NW * 16,), jnp.float32),
        mesh=mesh,
        scratch_types=[
            pltpu.VMEM((EPW + 16,), jnp.int32),
            pltpu.VMEM((ROWS, 128), jnp.int32),
            pltpu.VMEM((2 * HALF, 128), jnp.float32),
            pltpu.VMEM((SE_PW,), jnp.int32),
            pltpu.VMEM((SE_PW,), jnp.int32),
            pltpu.VMEM((SE_PW,), jnp.float32),
            pltpu.VMEM((SE_PW,), jnp.float32),
            pltpu.VMEM((SE_PW,), jnp.int32),
            pltpu.VMEM((SE_PW,), jnp.int32),
            pltpu.VMEM((DIAG_PW,), jnp.int32),
            pltpu.VMEM((DIAG_PW,), jnp.float32),
            pltpu.VMEM((16,), jnp.float32),
            pltpu.VMEM((16,), jnp.float32),
            pltpu.VMEM((16,), jnp.float32),
            pltpu.SemaphoreType.DMA,
            pltpu.SemaphoreType.DMA,
            pltpu.SemaphoreType.DMA,
        ],
    )
    return grid_kernel(bg, start, end, s)


def kernel(bigram, start, end, bigram_bias, samples):
    big = _tc_add(bigram, bigram_bias)
    partials = _sc_loss(big.reshape(-1), start, end,
                        samples.astype(jnp.int32).reshape(-1))
    loss = jnp.sum(partials)
    return (loss, 0)


# speedup vs baseline: 1.0698x; 1.0603x over previous
import functools

import jax
import jax.numpy as jnp
from jax import lax
from jax.experimental import pallas as pl
from jax.experimental.pallas import tpu as pltpu
from jax.experimental.pallas import tpu_sc as plsc

N_WORDS = 2048
N_SAMPLES = 4096
PATH_LEN = 256

NC = 2
NS = 16
NW = NC * NS

EPW = (N_SAMPLES * PATH_LEN) // NW
ROWS = EPW // 128
VPR = 8
HALF = 8
SUPER = ROWS // 16
SE_PW = N_SAMPLES // NW
SROWS = SE_PW
SROWS = SE_PW
DIAG_PW = N_WORDS // NW

_INV = 1.0 / N_SAMPLES


def _add_body(x_ref, y_ref, o_ref):
    o_ref[...] = (x_ref[...] + y_ref[...]).reshape(128, 16, 128)


def _tc_add(x, y):
    return pl.pallas_call(
        _add_body,
        grid=(16,),
        in_specs=[pl.BlockSpec((128, N_WORDS), lambda i: (i, 0))] * 2,
        out_specs=pl.BlockSpec((128, 16, 128), lambda i: (i, 0, 0)),
        out_shape=jax.ShapeDtypeStruct((N_WORDS, 16, 128), jnp.float32),
    )(x, y)


def _sc_body(bg_hbm, start_hbm, end_hbm, s_hbm,
             out_hbm,
             sv, idx_v, gb_v, s0i_v, e0i_v, sg_v, eg_v, ps_v, pe_v,
             di_v, dg_v, w0_v, w1_v, o_v,
             semA, semB, semT):
    wid = lax.axis_index("s") * NC + lax.axis_index("c")
    iota = lax.iota(jnp.int32, 16)
    zeros16 = jnp.zeros((16,), jnp.int32)

    pltpu.sync_copy(s_hbm.at[pl.ds(wid * EPW, EPW)], sv.at[pl.ds(0, EPW)])
    sv[pl.ds(EPW, 16)] = zeros16

    def idx_body(r, carry):
        for c in range(VPR):
            base = r * 128 + c * 16
            va = sv[pl.ds(base, 16)]
            vb = sv[pl.ds(base + 1, 16)]
            idx_v[r, pl.ds(c * 16, 16)] = va * N_WORDS + vb
        return carry
    lax.fori_loop(0, ROWS, idx_body, 0)

    def fire(row, slot, sem):
        pltpu.async_copy(bg_hbm.at[idx_v.at[row]], gb_v.at[slot], sem)

    def drain_half(sem, lo):
        for k in range(HALF):
            pltpu.make_async_copy(bg_hbm.at[pl.ds(0, 128)], gb_v.at[lo + k], sem).wait()

    for j in range(SROWS // 16):
        pos = wid * EPW + (j * 16 + iota) * PATH_LEN
        ps_v[pl.ds(j * 16, 16)] = pos
        pe_v[pl.ds(j * 16, 16)] = pos + (PATH_LEN - 1)
    for j in range(DIAG_PW // 16):
        i_vec = wid * DIAG_PW + j * 16 + iota
        valid = i_vec < N_WORDS - 1
        di_v[pl.ds(j * 16, 16)] = jnp.where(valid, i_vec * (N_WORDS + 1) + 1, 0)
    pltpu.async_copy(s_hbm.at[ps_v], s0i_v, semT)
    pltpu.async_copy(s_hbm.at[pe_v], e0i_v, semT)
    pltpu.async_copy(bg_hbm.at[di_v], dg_v, semT)
    pltpu.async_copy(start_hbm.at[zeros16], w0_v, semT)
    pltpu.async_copy(end_hbm.at[zeros16 + (N_WORDS - 1)], w1_v, semT)

    for k in range(HALF):
        fire(k, k, semA)
    for k in range(HALF):
        fire(HALF + k, HALF + k, semB)

    mask7 = jnp.where(iota < 15, 1.0, 0.0).astype(jnp.float32)

    def acc_slot(acc, k):
        for c in range(VPR - 1):
            acc = acc + gb_v[k, pl.ds(c * 16, 16)]
        last = gb_v[k, pl.ds(112, 16)]
        if k % 2 == 1:
            last = last * mask7
        return acc + last

    def gather_body(g, acc):
        drain_half(semA, 0)
        for k in range(HALF):
            acc = acc_slot(acc, k)

        @pl.when(g < SUPER - 1)
        def _():
            for k in range(HALF):
                fire((g + 1) * 16 + k, k, semA)

        drain_half(semB, HALF)
        for k in range(HALF):
            acc = acc_slot(acc, HALF + k)

        @pl.when(g < SUPER - 1)
        def _():
            for k in range(HALF):
                fire((g + 1) * 16 + 8 + k, HALF + k, semB)

        return acc

    acc = lax.fori_loop(0, SUPER, gather_body, jnp.zeros((16,), jnp.float32))

    pltpu.make_async_copy(start_hbm.at[pl.ds(0, SE_PW)], s0i_v, semT).wait()
    pltpu.make_async_copy(start_hbm.at[pl.ds(0, SE_PW)], e0i_v, semT).wait()
    pltpu.make_async_copy(bg_hbm.at[pl.ds(0, DIAG_PW)], dg_v, semT).wait()
    pltpu.make_async_copy(start_hbm.at[pl.ds(0, 16)], w0_v, semT).wait()
    pltpu.make_async_copy(end_hbm.at[pl.ds(0, 16)], w1_v, semT).wait()
    h1 = pltpu.async_copy(start_hbm.at[s0i_v], sg_v, semA)
    h2 = pltpu.async_copy(end_hbm.at[e0i_v], eg_v, semB)
    h1.wait()
    h2.wait()
    for c in range(SE_PW // 16):
        acc = acc + sg_v[pl.ds(c * 16, 16)] + eg_v[pl.ds(c * 16, 16)]

    accn = jnp.zeros((16,), jnp.float32)
    for j in range(DIAG_PW // 16):
        i_vec = wid * DIAG_PW + j * 16 + iota
        m = jnp.where(i_vec < N_WORDS - 1, 1.0, 0.0).astype(jnp.float32)
        accn = accn + dg_v[pl.ds(j * 16, 16)] * m

    lane0 = jnp.where(iota == 0, 1.0, 0.0).astype(jnp.float32)
    wmask = jnp.where(wid == 0, 1.0, 0.0).astype(jnp.float32)
    accn = accn + (w0_v[...] + w1_v[...]) * lane0 * wmask

    o_v[...] = acc * _INV - accn
    pltpu.sync_copy(o_v, out_hbm.at[pl.ds(wid * 16, 16)])


@jax.jit
def _sc_loss(bg, start, end, s):
    mesh = plsc.VectorSubcoreMesh(core_axis_name="c", subcore_axis_name="s",
                                  num_cores=NC, num_subcores=NS)
    grid_kernel = pl.kernel(
        _sc_body,
        out_type=jax.ShapeDtypeStruct((NW * 16,), jnp.float32),
        mesh=mesh,
        scratch_types=[
            pltpu.VMEM((EPW + 16,), jnp.int32),
            pltpu.VMEM((ROWS, 128), jnp.int32),
            pltpu.VMEM((2 * HALF, 128), jnp.float32),
            pltpu.VMEM((SE_PW,), jnp.int32),
            pltpu.VMEM((SE_PW,), jnp.int32),
            pltpu.VMEM((SE_PW,), jnp.float32),
            pltpu.VMEM((SE_PW,), jnp.float32),
            pltpu.VMEM((SE_PW,), jnp.int32),
            pltpu.VMEM((SE_PW,), jnp.int32),
            pltpu.VMEM((DIAG_PW,), jnp.int32),
            pltpu.VMEM((DIAG_PW,), jnp.float32),
            pltpu.VMEM((16,), jnp.float32),
            pltpu.VMEM((16,), jnp.float32),
            pltpu.VMEM((16,), jnp.float32),
            pltpu.SemaphoreType.DMA,
            pltpu.SemaphoreType.DMA,
            pltpu.SemaphoreType.DMA,
        ],
    )
    return grid_kernel(bg, start, end, s)


def kernel(bigram, start, end, bigram_bias, samples):
    big = _tc_add(bigram, bigram_bias)
    partials = _sc_loss(big.reshape(-1), start, end,
                        samples.astype(jnp.int32).reshape(-1))
    loss = jnp.sum(partials)
    return (loss, 0)


# speedup vs baseline: 1.0714x; 1.0015x over previous
import jax
import jax.numpy as jnp
from jax import lax
from jax.experimental import pallas as pl
from jax.experimental.pallas import tpu as pltpu
from jax.experimental.pallas import tpu_sc as plsc

N_WORDS = 2048
N_SAMPLES = 4096
PATH_LEN = 256

NC = 2
NS = 16
NW = NC * NS

EPW = (N_SAMPLES * PATH_LEN) // NW
ROWS = EPW // 128
VPR = 8
HALF = 8
SUPER = ROWS // 16
SE_PW = N_SAMPLES // NW
SROWS = SE_PW
DIAG_PW = N_WORDS // NW

_INV = 1.0 / N_SAMPLES


def _add_body(x_ref, y_ref, o_ref):
    o_ref[...] = (x_ref[...] + y_ref[...]).reshape(128, 16, 128)


def _tc_add(x, y):
    return pl.pallas_call(
        _add_body,
        grid=(16,),
        in_specs=[pl.BlockSpec((128, N_WORDS), lambda i: (i, 0))] * 2,
        out_specs=pl.BlockSpec((128, 16, 128), lambda i: (i, 0, 0)),
        out_shape=jax.ShapeDtypeStruct((N_WORDS, 16, 128), jnp.float32),
    )(x, y)


def _sc_body(bg_hbm, start_hbm, end_hbm, s_hbm,
             out_hbm,
             sv, idx_v, gb_v, s0i_v, e0i_v, sg_v, eg_v, ps_v, pe_v,
             di_v, dg_v, w0_v, w1_v, o_v,
             semA, semB, semT):
    wid = lax.axis_index("s") * NC + lax.axis_index("c")
    iota = lax.iota(jnp.int32, 16)
    zeros16 = jnp.zeros((16,), jnp.int32)

    pltpu.sync_copy(s_hbm.at[pl.ds(wid * EPW, EPW)], sv.at[pl.ds(0, EPW)])
    sv[pl.ds(EPW, 16)] = zeros16

    def idx_body(r, carry):
        for c in range(VPR):
            base = r * 128 + c * 16
            va = sv[pl.ds(base, 16)]
            vb = sv[pl.ds(base + 1, 16)]
            idx_v[r, pl.ds(c * 16, 16)] = va * N_WORDS + vb
        return carry
    lax.fori_loop(0, ROWS, idx_body, 0)

    def fire(row, slot, sem):
        pltpu.async_copy(bg_hbm.at[idx_v.at[row]], gb_v.at[slot], sem)

    def drain_half(sem, lo):
        for k in range(HALF):
            pltpu.make_async_copy(bg_hbm.at[pl.ds(0, 128)], gb_v.at[lo + k], sem).wait()

    for j in range(SROWS // 16):
        pos = wid * EPW + (j * 16 + iota) * PATH_LEN
        ps_v[pl.ds(j * 16, 16)] = pos
        pe_v[pl.ds(j * 16, 16)] = pos + (PATH_LEN - 1)
    for j in range(DIAG_PW // 16):
        i_vec = wid * DIAG_PW + j * 16 + iota
        valid = i_vec < N_WORDS - 1
        di_v[pl.ds(j * 16, 16)] = jnp.where(valid, i_vec * (N_WORDS + 1) + 1, 0)
    pltpu.async_copy(s_hbm.at[ps_v], s0i_v, semT)
    pltpu.async_copy(s_hbm.at[pe_v], e0i_v, semT)
    pltpu.async_copy(bg_hbm.at[di_v], dg_v, semT)
    pltpu.async_copy(start_hbm.at[zeros16], w0_v, semT)
    pltpu.async_copy(end_hbm.at[zeros16 + (N_WORDS - 1)], w1_v, semT)

    for k in range(HALF):
        fire(k, k, semA)
    for k in range(HALF):
        fire(HALF + k, HALF + k, semB)

    mask7 = jnp.where(iota < 15, 1.0, 0.0).astype(jnp.float32)

    def acc_slot(acc, k):
        for c in range(VPR - 1):
            acc = acc + gb_v[k, pl.ds(c * 16, 16)]
        last = gb_v[k, pl.ds(112, 16)]
        if k % 2 == 1:
            last = last * mask7
        return acc + last

    def gather_body(g, acc):
        drain_half(semA, 0)
        for k in range(HALF):
            acc = acc_slot(acc, k)

        @pl.when(g < SUPER - 1)
        def _():
            for k in range(HALF):
                fire((g + 1) * 16 + k, k, semA)

        drain_half(semB, HALF)
        for k in range(HALF):
            acc = acc_slot(acc, HALF + k)

        @pl.when(g < SUPER - 1)
        def _():
            for k in range(HALF):
                fire((g + 1) * 16 + 8 + k, HALF + k, semB)

        return acc

    acc = lax.fori_loop(0, SUPER, gather_body, jnp.zeros((16,), jnp.float32))

    pltpu.make_async_copy(start_hbm.at[pl.ds(0, SE_PW)], s0i_v, semT).wait()
    pltpu.make_async_copy(start_hbm.at[pl.ds(0, SE_PW)], e0i_v, semT).wait()
    pltpu.make_async_copy(bg_hbm.at[pl.ds(0, DIAG_PW)], dg_v, semT).wait()
    pltpu.make_async_copy(start_hbm.at[pl.ds(0, 16)], w0_v, semT).wait()
    pltpu.make_async_copy(end_hbm.at[pl.ds(0, 16)], w1_v, semT).wait()
    h1 = pltpu.async_copy(start_hbm.at[s0i_v], sg_v, semA)
    h2 = pltpu.async_copy(end_hbm.at[e0i_v], eg_v, semB)
    h1.wait()
    h2.wait()
    for c in range(SE_PW // 16):
        acc = acc + sg_v[pl.ds(c * 16, 16)] + eg_v[pl.ds(c * 16, 16)]

    accn = jnp.zeros((16,), jnp.float32)
    for j in range(DIAG_PW // 16):
        i_vec = wid * DIAG_PW + j * 16 + iota
        m = jnp.where(i_vec < N_WORDS - 1, 1.0, 0.0).astype(jnp.float32)
        accn = accn + dg_v[pl.ds(j * 16, 16)] * m

    lane0 = jnp.where(iota == 0, 1.0, 0.0).astype(jnp.float32)
    wmask = jnp.where(wid == 0, 1.0, 0.0).astype(jnp.float32)
    accn = accn + (w0_v[...] + w1_v[...]) * lane0 * wmask

    o_v[...] = acc * _INV - accn
    pltpu.sync_copy(o_v, out_hbm.at[pl.ds(wid * 16, 16)])


@jax.jit
def _sc_loss(bg, start, end, s):
    mesh = plsc.VectorSubcoreMesh(core_axis_name="c", subcore_axis_name="s",
                                  num_cores=NC, num_subcores=NS)
    grid_kernel = pl.kernel(
        _sc_body,
        out_type=jax.ShapeDtypeStruct((NW * 16,), jnp.float32),
        mesh=mesh,
        scratch_types=[
            pltpu.VMEM((EPW + 16,), jnp.int32),
            pltpu.VMEM((ROWS, 128), jnp.int32),
            pltpu.VMEM((2 * HALF, 128), jnp.float32),
            pltpu.VMEM((SE_PW,), jnp.int32),
            pltpu.VMEM((SE_PW,), jnp.int32),
            pltpu.VMEM((SE_PW,), jnp.float32),
            pltpu.VMEM((SE_PW,), jnp.float32),
            pltpu.VMEM((SE_PW,), jnp.int32),
            pltpu.VMEM((SE_PW,), jnp.int32),
            pltpu.VMEM((DIAG_PW,), jnp.int32),
            pltpu.VMEM((DIAG_PW,), jnp.float32),
            pltpu.VMEM((16,), jnp.float32),
            pltpu.VMEM((16,), jnp.float32),
            pltpu.VMEM((16,), jnp.float32),
            pltpu.SemaphoreType.DMA,
            pltpu.SemaphoreType.DMA,
            pltpu.SemaphoreType.DMA,
        ],
    )
    return grid_kernel(bg, start, end, s)


def kernel(bigram, start, end, bigram_bias, samples):
    big = _tc_add(bigram, bigram_bias)
    partials = _sc_loss(big.reshape(-1), start, end,
                        samples.astype(jnp.int32).reshape(-1))
    loss = jnp.sum(partials)
    return (loss, 0)
